# Initial kernel scaffold; baseline (speedup 1.0000x reference)
import jax
import jax.numpy as jnp
from jax.experimental import pallas as pl


def kernel(x, p, pos_table):
    raise NotImplementedError("write your pallas kernel here")


# speedup vs baseline: 11.0523x; 11.0523x over previous
import functools

import jax
import jax.numpy as jnp
from jax import lax
from jax.experimental import pallas as pl
from jax.experimental.pallas import tpu as pltpu
from jax.experimental.pallas import tpu_sc as plsc

N_ROWS = 16384
ROW_W = 3200
N_IDX_PER_ROW = 200
EMBED = 16
N_G = N_ROWS * N_IDX_PER_ROW

NC = 2
NS = 16
NW = NC * NS

G_PER_W = N_G // NW
BLK = 1600
N_BLK = G_PER_W // BLK
CHUNK = 128
N_FULL_CHUNKS = BLK // CHUNK
TAIL = BLK - N_FULL_CHUNKS * CHUNK


def _sc_body(x_hbm, p_hbm, tab_hbm, out_hbm, idx_v, pe_v, x_v, sem):
    wid = lax.axis_index("s") * NC + lax.axis_index("c")
    g0 = wid * G_PER_W

    def blk(b, carry):
        base = g0 + b * BLK
        pltpu.sync_copy(p_hbm.at[pl.ds(base, BLK)], idx_v)
        pltpu.sync_copy(x_hbm.at[pl.ds(base, BLK), :], x_v)
        copies = []
        for c in range(N_FULL_CHUNKS):
            copies.append(pltpu.async_copy(
                tab_hbm.at[idx_v.at[pl.ds(c * CHUNK, CHUNK)]],
                pe_v.at[pl.ds(c * CHUNK, CHUNK), :],
                sem,
            ))
        copies.append(pltpu.async_copy(
            tab_hbm.at[idx_v.at[pl.ds(N_FULL_CHUNKS * CHUNK, TAIL)]],
            pe_v.at[pl.ds(N_FULL_CHUNKS * CHUNK, TAIL), :],
            sem,
        ))
        for cp in copies:
            cp.wait()

        def add_i(i, c2):
            plsc.addupdate(x_v.at[i, :], pe_v[i, :])
            return c2

        lax.fori_loop(0, BLK, add_i, 0, unroll=4)
        pltpu.sync_copy(x_v, out_hbm.at[pl.ds(base, BLK), :])
        return carry

    lax.fori_loop(0, N_BLK, blk, 0)


@functools.partial(jax.jit, static_argnames=())
def _run(x16, pf, tab):
    mesh = plsc.VectorSubcoreMesh(
        core_axis_name="c", subcore_axis_name="s", num_cores=NC,
        num_subcores=NS,
    )
    return pl.kernel(
        _sc_body,
        out_type=jax.ShapeDtypeStruct((N_G, EMBED), jnp.float32),
        mesh=mesh,
        scratch_types=[
            pltpu.VMEM((BLK,), jnp.int32),
            pltpu.VMEM((BLK, EMBED), jnp.float32),
            pltpu.VMEM((BLK, EMBED), jnp.float32),
            pltpu.SemaphoreType.DMA,
        ],
        compiler_params=pltpu.CompilerParams(use_tc_tiling_on_sc=False),
    )(x16, pf, tab)


def kernel(x, p, pos_table):
    x16 = x.reshape(N_G, EMBED)
    pf = p.reshape(N_G).astype(jnp.int32)
    out = _run(x16, pf, pos_table)
    return out.reshape(x.shape)


# speedup vs baseline: 15.8894x; 1.4377x over previous
import functools

import jax
import jax.numpy as jnp
from jax import lax
from jax.experimental import pallas as pl
from jax.experimental.pallas import tpu as pltpu
from jax.experimental.pallas import tpu_sc as plsc

N_ROWS = 16384
ROW_W = 3200
N_IDX_PER_ROW = 200
EMBED = 16
N_G = N_ROWS * N_IDX_PER_ROW

NC = 2
NS = 16
NW = NC * NS

G_PER_W = N_G // NW
BLK = 1600
N_BLK = G_PER_W // BLK
CHUNK = 128
N_FULL_CHUNKS = BLK // CHUNK
TAIL = BLK - N_FULL_CHUNKS * CHUNK


def _sc_body(x_hbm, p_hbm, tab_hbm, out_hbm, idx_v, pe_v, x_v, sem):
    wid = lax.axis_index("s") * NC + lax.axis_index("c")
    g0 = wid * G_PER_W

    def blk(b, carry):
        base = g0 + b * BLK
        pltpu.sync_copy(p_hbm.at[pl.ds(base, BLK)], idx_v)
        pltpu.sync_copy(x_hbm.at[pl.ds(base, BLK), :], x_v)
        copies = []
        for c in range(N_FULL_CHUNKS):
            copies.append(pltpu.async_copy(
                tab_hbm.at[idx_v.at[pl.ds(c * CHUNK, CHUNK)]],
                x_v.at[pl.ds(c * CHUNK, CHUNK), :],
                sem,
                add=True,
            ))
        copies.append(pltpu.async_copy(
            tab_hbm.at[idx_v.at[pl.ds(N_FULL_CHUNKS * CHUNK, TAIL)]],
            x_v.at[pl.ds(N_FULL_CHUNKS * CHUNK, TAIL), :],
            sem,
            add=True,
        ))
        for cp in copies:
            cp.wait()
        pltpu.sync_copy(x_v, out_hbm.at[pl.ds(base, BLK), :])
        return carry

    lax.fori_loop(0, N_BLK, blk, 0)


@functools.partial(jax.jit, static_argnames=())
def _run(x16, pf, tab):
    mesh = plsc.VectorSubcoreMesh(
        core_axis_name="c", subcore_axis_name="s", num_cores=NC,
        num_subcores=NS,
    )
    return pl.kernel(
        _sc_body,
        out_type=jax.ShapeDtypeStruct((N_G, EMBED), jnp.float32),
        mesh=mesh,
        scratch_types=[
            pltpu.VMEM((BLK,), jnp.int32),
            pltpu.VMEM((BLK, EMBED), jnp.float32),
            pltpu.VMEM((BLK, EMBED), jnp.float32),
            pltpu.SemaphoreType.DMA,
        ],
        compiler_params=pltpu.CompilerParams(use_tc_tiling_on_sc=False),
    )(x16, pf, tab)


def kernel(x, p, pos_table):
    x16 = x.reshape(N_G, EMBED)
    pf = p.reshape(N_G).astype(jnp.int32)
    out = _run(x16, pf, pos_table)
    return out.reshape(x.shape)


# speedup vs baseline: 17.9740x; 1.1312x over previous
import functools

import jax
import jax.numpy as jnp
from jax import lax
from jax.experimental import pallas as pl
from jax.experimental.pallas import tpu as pltpu
from jax.experimental.pallas import tpu_sc as plsc

N_ROWS = 16384
ROW_W = 3200
N_IDX_PER_ROW = 200
EMBED = 16
N_G = N_ROWS * N_IDX_PER_ROW

NC = 2
NS = 16
NW = NC * NS

G_PER_W = N_G // NW
BLK = 1600
N_BLK = G_PER_W // BLK
CHUNK = 128
N_FULL_CHUNKS = BLK // CHUNK
TAIL = BLK - N_FULL_CHUNKS * CHUNK
NBUF = 4
PREF = 2


def _gather_add(tab_hbm, idx_b, x_b, sem):
    copies = []
    for c in range(N_FULL_CHUNKS):
        copies.append(pltpu.async_copy(
            tab_hbm.at[idx_b.at[pl.ds(c * CHUNK, CHUNK)]],
            x_b.at[pl.ds(c * CHUNK, CHUNK), :],
            sem,
            add=True,
        ))
    copies.append(pltpu.async_copy(
        tab_hbm.at[idx_b.at[pl.ds(N_FULL_CHUNKS * CHUNK, TAIL)]],
        x_b.at[pl.ds(N_FULL_CHUNKS * CHUNK, TAIL), :],
        sem,
        add=True,
    ))
    for cp in copies:
        cp.wait()


def _sc_body(x_hbm, p_hbm, tab_hbm, out_hbm, idx_v, x_v, sem_ld, sem_st,
             sem_g):
    wid = lax.axis_index("s") * NC + lax.axis_index("c")
    g0 = wid * G_PER_W

    def start_loads(blk_i, j):
        base = g0 + blk_i * BLK
        pltpu.async_copy(p_hbm.at[pl.ds(base, BLK)], idx_v.at[j],
                         sem_ld.at[j])
        pltpu.async_copy(x_hbm.at[pl.ds(base, BLK), :], x_v.at[j],
                         sem_ld.at[j])

    def wait_loads(blk_i, j):
        base = g0 + blk_i * BLK
        pltpu.make_async_copy(p_hbm.at[pl.ds(base, BLK)], idx_v.at[j],
                              sem_ld.at[j]).wait()
        pltpu.make_async_copy(x_hbm.at[pl.ds(base, BLK), :], x_v.at[j],
                              sem_ld.at[j]).wait()

    def wait_store(blk_i, j):
        base = g0 + blk_i * BLK
        pltpu.make_async_copy(x_v.at[j], out_hbm.at[pl.ds(base, BLK), :],
                              sem_st.at[j]).wait()

    for j in range(PREF):
        start_loads(j, j)

    @pl.loop(0, N_BLK, step=NBUF)
    def blk_loop(b0):
        for j in range(NBUF):
            b = b0 + j
            nb = b + PREF
            jn = (j + PREF) % NBUF

            @pl.when(nb < N_BLK)
            def _prefetch():
                @pl.when(b >= PREF)
                def _drain_store():
                    wait_store(b - PREF, jn)
                start_loads(nb, jn)

            wait_loads(b, j)
            _gather_add(tab_hbm, idx_v.at[j], x_v.at[j], sem_g)
            base = g0 + b * BLK
            pltpu.async_copy(x_v.at[j], out_hbm.at[pl.ds(base, BLK), :],
                             sem_st.at[j])

    for b in range(N_BLK - NBUF, N_BLK):
        wait_store(b, b % NBUF)


@functools.partial(jax.jit, static_argnames=())
def _run(x16, pf, tab):
    mesh = plsc.VectorSubcoreMesh(
        core_axis_name="c", subcore_axis_name="s", num_cores=NC,
        num_subcores=NS,
    )
    return pl.kernel(
        _sc_body,
        out_type=jax.ShapeDtypeStruct((N_G, EMBED), jnp.float32),
        mesh=mesh,
        scratch_types=[
            pltpu.VMEM((NBUF, BLK), jnp.int32),
            pltpu.VMEM((NBUF, BLK, EMBED), jnp.float32),
            pltpu.SemaphoreType.DMA((NBUF,)),
            pltpu.SemaphoreType.DMA((NBUF,)),
            pltpu.SemaphoreType.DMA,
        ],
        compiler_params=pltpu.CompilerParams(use_tc_tiling_on_sc=False),
    )(x16, pf, tab)


def kernel(x, p, pos_table):
    x16 = x.reshape(N_G, EMBED)
    pf = p.reshape(N_G).astype(jnp.int32)
    out = _run(x16, pf, pos_table)
    return out.reshape(x.shape)
